# baseline (speedup 1.0000x reference)
import functools

import jax
import jax.numpy as jnp
from jax import lax
from jax.experimental import pallas as pl
from jax.experimental.pallas import tpu as pltpu
from jax.experimental.pallas import tpu_sc as plsc

N = 10000
NDP = 10240
E = 320000
D = 128

NC = 2
NS = 16
NW = NC * NS
EPT = E // NW
C = 80
NCHUNK = EPT // C
RPT = 624
RPTD = NDP // NS

BR = 2000
GRID = N // BR

_mesh = plsc.VectorSubcoreMesh(core_axis_name="c", subcore_axis_name="s")


@functools.partial(
    pl.kernel,
    out_type=jax.ShapeDtypeStruct((NC, 2, NDP), jnp.float32),
    mesh=_mesh,
    scratch_types=[
        pltpu.VMEM_SHARED((NDP,), jnp.float32),
        pltpu.VMEM_SHARED((NDP,), jnp.float32),
        pltpu.VMEM((NCHUNK, C), jnp.int32),
        pltpu.VMEM((NCHUNK, C), jnp.int32),
        pltpu.VMEM((C,), jnp.float32),
        pltpu.SemaphoreType.DMA,
        pltpu.SemaphoreType.DMA,
        pltpu.SemaphoreType.DMA,
        pltpu.SemaphoreType.DMA,
    ],
)
def _deg_kernel(src_hbm, dst_hbm, zeros_hbm, out_hbm, acc_o, acc_i, sidx, didx, ones,
                d0, d1, d2, d3):
    c = lax.axis_index("c")
    s = lax.axis_index("s")
    tile = c * NS + s
    z0 = pltpu.async_copy(zeros_hbm.at[pl.ds(s * RPTD, RPTD)], acc_o.at[pl.ds(s * RPTD, RPTD)], d0)
    z1 = pltpu.async_copy(zeros_hbm.at[pl.ds(s * RPTD, RPTD)], acc_i.at[pl.ds(s * RPTD, RPTD)], d1)
    i0 = pltpu.async_copy(src_hbm.at[tile], sidx, d2)
    i1 = pltpu.async_copy(dst_hbm.at[tile], didx, d3)
    for j in range(C // 16):
        ones[pl.ds(j * 16, 16)] = jnp.full((16,), 1.0, jnp.float32)
    z0.wait(); z1.wait(); i0.wait(); i1.wait()
    plsc.subcore_barrier()

    def body(j, _):
        k = 2 * j
        a0 = pltpu.async_copy(ones, acc_o.at[sidx.at[k]], d0, add=True)
        a1 = pltpu.async_copy(ones, acc_i.at[didx.at[k]], d1, add=True)
        a2 = pltpu.async_copy(ones, acc_o.at[sidx.at[k + 1]], d2, add=True)
        a3 = pltpu.async_copy(ones, acc_i.at[didx.at[k + 1]], d3, add=True)
        a0.wait(); a1.wait(); a2.wait(); a3.wait()
        return ()

    lax.fori_loop(0, NCHUNK // 2, body, ())
    la = pltpu.async_copy(ones, acc_o.at[sidx.at[NCHUNK - 1]], d0, add=True)
    lb = pltpu.async_copy(ones, acc_i.at[didx.at[NCHUNK - 1]], d1, add=True)
    la.wait(); lb.wait()
    plsc.subcore_barrier()
    pltpu.sync_copy(acc_o.at[pl.ds(s * RPTD, RPTD)], out_hbm.at[c, 0, pl.ds(s * RPTD, RPTD)])
    pltpu.sync_copy(acc_i.at[pl.ds(s * RPTD, RPTD)], out_hbm.at[c, 1, pl.ds(s * RPTD, RPTD)])


@functools.partial(
    pl.kernel,
    out_type=jax.ShapeDtypeStruct((NC, N, D), jnp.float32),
    mesh=_mesh,
    scratch_types=[
        pltpu.VMEM_SHARED((N, D), jnp.float32),
        pltpu.VMEM((EPT,), jnp.int32),
        pltpu.VMEM((NCHUNK, C), jnp.int32),
        pltpu.VMEM((C, D), jnp.float32),
        pltpu.VMEM((C, D), jnp.float32),
        pltpu.SemaphoreType.DMA,
        pltpu.SemaphoreType.DMA,
    ],
)
def _seg_kernel(h_hbm, src_hbm, dst_hbm, zrows_hbm, out_hbm, acc, sidx, didx,
                rows0, rows1, gs0, gs1):
    c = lax.axis_index("c")
    s = lax.axis_index("s")
    tile = c * NS + s
    z0 = pltpu.async_copy(zrows_hbm.at[pl.ds(s * RPT, RPT)], acc.at[pl.ds(s * RPT, RPT)], gs0)
    i0 = pltpu.async_copy(src_hbm.at[tile], sidx, gs1)
    i1 = pltpu.async_copy(dst_hbm.at[tile], didx, gs1)

    @pl.when(s == 0)
    def _zero_tail():
        pltpu.async_copy(zrows_hbm.at[pl.ds(NS * RPT, N - NS * RPT)],
                         acc.at[pl.ds(NS * RPT, N - NS * RPT)], gs0).wait()

    z0.wait(); i0.wait(); i1.wait()
    plsc.subcore_barrier()

    def _gwait(buf, sem):
        pltpu.make_async_copy(h_hbm.at[pl.ds(0, C)], buf, sem).wait()

    pltpu.async_copy(h_hbm.at[sidx.at[pl.ds(0, C)]], rows0, gs0)

    def body(j, _):
        k = 2 * j
        pltpu.async_copy(h_hbm.at[sidx.at[pl.ds(pl.multiple_of((k + 1) * C, 8), C)]], rows1, gs1)
        _gwait(rows0, gs0)
        pltpu.sync_copy(rows0, acc.at[didx.at[k]], add=True)
        pltpu.async_copy(h_hbm.at[sidx.at[pl.ds(pl.multiple_of((k + 2) * C, 8), C)]], rows0, gs0)
        _gwait(rows1, gs1)
        pltpu.sync_copy(rows1, acc.at[didx.at[k + 1]], add=True)
        return ()

    lax.fori_loop(0, (NCHUNK - 1) // 2, body, ())
    _gwait(rows0, gs0)
    pltpu.sync_copy(rows0, acc.at[didx.at[NCHUNK - 1]], add=True)
    plsc.subcore_barrier()
    pltpu.sync_copy(acc.at[pl.ds(s * RPT, RPT)], out_hbm.at[c, pl.ds(s * RPT, RPT)])

    @pl.when(s == 0)
    def _out_tail():
        pltpu.sync_copy(acc.at[pl.ds(NS * RPT, N - NS * RPT)],
                        out_hbm.at[c, pl.ds(NS * RPT, N - NS * RPT)])


def _mm0a_body(x_ref, w_ref, h_ref):
    h_ref[...] = jnp.dot(x_ref[...], w_ref[...], preferred_element_type=jnp.float32)


def _mm0b_body(h_ref, deg_ref, ho_ref, dis_ref):
    deg = deg_ref[...]
    deg_out = deg[:, 0:1] + deg[:, 2:3]
    deg_in = deg[:, 1:2] + deg[:, 3:4]
    dis_out = jnp.where(deg_out > 0, lax.rsqrt(jnp.maximum(deg_out, 1.0)), 0.0)
    dis_in = jnp.where(deg_in > 0, lax.rsqrt(jnp.maximum(deg_in, 1.0)), 0.0)
    ho_ref[...] = h_ref[...] * dis_out
    dis_ref[...] = jnp.concatenate([dis_out, dis_in], axis=1)


def _mm_mid_body(p0_ref, p1_ref, dis_ref, w_ref, b_ref, h_ref):
    dis = dis_ref[...]
    agg = (p0_ref[...] + p1_ref[...]) * dis[:, 1:2]
    t = jnp.maximum(agg + b_ref[...], 0.0)
    h = jnp.dot(t, w_ref[...], preferred_element_type=jnp.float32)
    h_ref[...] = h * dis[:, 0:1]


def _mm_fin_body(p0_ref, p1_ref, dis_ref, w_ref, b_ref, bfc_ref, o_ref):
    dis = dis_ref[...]
    agg = (p0_ref[...] + p1_ref[...]) * dis[:, 1:2]
    t = jnp.maximum(agg + b_ref[...], 0.0)
    o_ref[...] = jnp.dot(t, w_ref[...], preferred_element_type=jnp.float32) + bfc_ref[...]


_row_spec = pl.BlockSpec((BR, D), lambda i: (i, 0))
_w_spec = pl.BlockSpec((D, D), lambda i: (0, 0))
_b_spec = pl.BlockSpec((1, D), lambda i: (0, 0))
_dis_spec = pl.BlockSpec((BR, 2), lambda i: (i, 0))
_deg_spec = pl.BlockSpec((BR, 4), lambda i: (i, 0))

_mm0a = pl.pallas_call(
    _mm0a_body,
    grid=(GRID,),
    in_specs=[_row_spec, _w_spec],
    out_specs=_row_spec,
    out_shape=jax.ShapeDtypeStruct((N, D), jnp.float32),
)

_mm0b = pl.pallas_call(
    _mm0b_body,
    grid=(GRID,),
    in_specs=[_row_spec, _deg_spec],
    out_specs=[_row_spec, _dis_spec],
    out_shape=[
        jax.ShapeDtypeStruct((N, D), jnp.float32),
        jax.ShapeDtypeStruct((N, 2), jnp.float32),
    ],
)

_mm_mid = pl.pallas_call(
    _mm_mid_body,
    grid=(GRID,),
    in_specs=[_row_spec, _row_spec, _dis_spec, _w_spec, _b_spec],
    out_specs=_row_spec,
    out_shape=jax.ShapeDtypeStruct((N, D), jnp.float32),
)

_mm_fin = pl.pallas_call(
    _mm_fin_body,
    grid=(GRID,),
    in_specs=[_row_spec, _row_spec, _dis_spec, _w_spec, _b_spec, _b_spec],
    out_specs=_row_spec,
    out_shape=jax.ShapeDtypeStruct((N, D), jnp.float32),
)


def kernel(features, edge_index, W0, b0, W1, b1, W2, b2, Wfc, bfc):
    src3 = edge_index[0].reshape(NW, NCHUNK, C)
    src2 = edge_index[0].reshape(NW, EPT)
    dst3 = edge_index[1].reshape(NW, NCHUNK, C)
    zn = jnp.zeros((NDP,), jnp.float32)
    zrows = jnp.zeros((N, D), jnp.float32)
    b0r, b1r, b2r, bfcr = (v.reshape(1, D) for v in (b0, b1, b2, bfc))

    deg_parts = _deg_kernel(src3, dst3, zn)
    hraw = _mm0a(features, W0)
    deg4 = jnp.transpose(deg_parts.reshape(4, NDP))[:N]
    h, dis = _mm0b(hraw, deg4)
    p = _seg_kernel(h, src2, dst3, zrows)
    h = _mm_mid(p[0], p[1], dis, W1, b0r)
    p = _seg_kernel(h, src2, dst3, zrows)
    h = _mm_mid(p[0], p[1], dis, W2, b1r)
    p = _seg_kernel(h, src2, dst3, zrows)
    return _mm_fin(p[0], p[1], dis, Wfc, b2r, bfcr)


# speedup vs baseline: 1.0747x; 1.0747x over previous
import functools

import jax
import jax.numpy as jnp
from jax import lax
from jax.experimental import pallas as pl
from jax.experimental.pallas import tpu as pltpu
from jax.experimental.pallas import tpu_sc as plsc

N = 10000
NDP = 10240
E = 320000
D = 128

NC = 2
NS = 16
NW = NC * NS
EPT = E // NW
C = 80
NCHUNK = EPT // C
RPT = 624
RPTD = NDP // NS

BR = 2000
GRID = N // BR

_mesh = plsc.VectorSubcoreMesh(core_axis_name="c", subcore_axis_name="s")


@functools.partial(
    pl.kernel,
    out_type=jax.ShapeDtypeStruct((NC, 2, NDP), jnp.float32),
    mesh=_mesh,
    scratch_types=[
        pltpu.VMEM_SHARED((NDP,), jnp.float32),
        pltpu.VMEM_SHARED((NDP,), jnp.float32),
        pltpu.VMEM((NCHUNK, C), jnp.int32),
        pltpu.VMEM((NCHUNK, C), jnp.int32),
        pltpu.VMEM((C,), jnp.float32),
        pltpu.SemaphoreType.DMA,
        pltpu.SemaphoreType.DMA,
        pltpu.SemaphoreType.DMA,
        pltpu.SemaphoreType.DMA,
    ],
)
def _deg_kernel(e4_hbm, zeros_hbm, out_hbm, acc_o, acc_i, sidx, didx, ones,
                d0, d1, d2, d3):
    c = lax.axis_index("c")
    s = lax.axis_index("s")
    tile = c * NS + s
    z0 = pltpu.async_copy(zeros_hbm.at[pl.ds(s * RPTD, RPTD)], acc_o.at[pl.ds(s * RPTD, RPTD)], d0)
    z1 = pltpu.async_copy(zeros_hbm.at[pl.ds(s * RPTD, RPTD)], acc_i.at[pl.ds(s * RPTD, RPTD)], d1)
    i0 = pltpu.async_copy(e4_hbm.at[0, tile], sidx, d2)
    i1 = pltpu.async_copy(e4_hbm.at[1, tile], didx, d3)
    for j in range(C // 16):
        ones[pl.ds(j * 16, 16)] = jnp.full((16,), 1.0, jnp.float32)
    z0.wait(); z1.wait(); i0.wait(); i1.wait()
    plsc.subcore_barrier()

    def body(j, _):
        k = 2 * j
        a0 = pltpu.async_copy(ones, acc_o.at[sidx.at[k]], d0, add=True)
        a1 = pltpu.async_copy(ones, acc_i.at[didx.at[k]], d1, add=True)
        a2 = pltpu.async_copy(ones, acc_o.at[sidx.at[k + 1]], d2, add=True)
        a3 = pltpu.async_copy(ones, acc_i.at[didx.at[k + 1]], d3, add=True)
        a0.wait(); a1.wait(); a2.wait(); a3.wait()
        return ()

    lax.fori_loop(0, NCHUNK // 2, body, ())
    la = pltpu.async_copy(ones, acc_o.at[sidx.at[NCHUNK - 1]], d0, add=True)
    lb = pltpu.async_copy(ones, acc_i.at[didx.at[NCHUNK - 1]], d1, add=True)
    la.wait(); lb.wait()
    plsc.subcore_barrier()
    pltpu.sync_copy(acc_o.at[pl.ds(s * RPTD, RPTD)], out_hbm.at[c, 0, pl.ds(s * RPTD, RPTD)])
    pltpu.sync_copy(acc_i.at[pl.ds(s * RPTD, RPTD)], out_hbm.at[c, 1, pl.ds(s * RPTD, RPTD)])


@functools.partial(
    pl.kernel,
    out_type=jax.ShapeDtypeStruct((NC, N, D), jnp.float32),
    mesh=_mesh,
    scratch_types=[
        pltpu.VMEM_SHARED((N, D), jnp.float32),
        pltpu.VMEM((EPT,), jnp.int32),
        pltpu.VMEM((NCHUNK, C), jnp.int32),
        pltpu.VMEM((C, D), jnp.float32),
        pltpu.VMEM((C, D), jnp.float32),
        pltpu.SemaphoreType.DMA,
        pltpu.SemaphoreType.DMA,
    ],
)
def _seg_kernel(h_hbm, e2_hbm, e4_hbm, zrows_hbm, out_hbm, acc, sidx, didx,
                rows0, rows1, gs0, gs1):
    c = lax.axis_index("c")
    s = lax.axis_index("s")
    tile = c * NS + s
    z0 = pltpu.async_copy(zrows_hbm.at[pl.ds(s * RPT, RPT)], acc.at[pl.ds(s * RPT, RPT)], gs0)
    i0 = pltpu.async_copy(e2_hbm.at[0, tile], sidx, gs1)
    i1 = pltpu.async_copy(e4_hbm.at[1, tile], didx, gs1)

    @pl.when(s == 0)
    def _zero_tail():
        pltpu.async_copy(zrows_hbm.at[pl.ds(NS * RPT, N - NS * RPT)],
                         acc.at[pl.ds(NS * RPT, N - NS * RPT)], gs0).wait()

    z0.wait(); i0.wait(); i1.wait()
    plsc.subcore_barrier()

    def _gwait(buf, sem):
        pltpu.make_async_copy(h_hbm.at[pl.ds(0, C)], buf, sem).wait()

    pltpu.async_copy(h_hbm.at[sidx.at[pl.ds(0, C)]], rows0, gs0)

    def body(j, _):
        k = 2 * j
        pltpu.async_copy(h_hbm.at[sidx.at[pl.ds(pl.multiple_of((k + 1) * C, 8), C)]], rows1, gs1)
        _gwait(rows0, gs0)
        pltpu.sync_copy(rows0, acc.at[didx.at[k]], add=True)
        pltpu.async_copy(h_hbm.at[sidx.at[pl.ds(pl.multiple_of((k + 2) * C, 8), C)]], rows0, gs0)
        _gwait(rows1, gs1)
        pltpu.sync_copy(rows1, acc.at[didx.at[k + 1]], add=True)
        return ()

    lax.fori_loop(0, (NCHUNK - 1) // 2, body, ())
    _gwait(rows0, gs0)
    pltpu.sync_copy(rows0, acc.at[didx.at[NCHUNK - 1]], add=True)
    plsc.subcore_barrier()
    pltpu.sync_copy(acc.at[pl.ds(s * RPT, RPT)], out_hbm.at[c, pl.ds(s * RPT, RPT)])

    @pl.when(s == 0)
    def _out_tail():
        pltpu.sync_copy(acc.at[pl.ds(NS * RPT, N - NS * RPT)],
                        out_hbm.at[c, pl.ds(NS * RPT, N - NS * RPT)])


def _mm0a_body(x_ref, w_ref, h_ref):
    h_ref[...] = jnp.dot(x_ref[...], w_ref[...], preferred_element_type=jnp.float32)


def _mm0b_body(h_ref, deg_ref, ho_ref, dis_ref):
    deg = deg_ref[...]
    deg_out = deg[:, 0:1] + deg[:, 2:3]
    deg_in = deg[:, 1:2] + deg[:, 3:4]
    dis_out = jnp.where(deg_out > 0, lax.rsqrt(jnp.maximum(deg_out, 1.0)), 0.0)
    dis_in = jnp.where(deg_in > 0, lax.rsqrt(jnp.maximum(deg_in, 1.0)), 0.0)
    ho_ref[...] = h_ref[...] * dis_out
    dis_ref[...] = jnp.concatenate([dis_out, dis_in], axis=1)


def _mm_mid_body(p_ref, dis_ref, w_ref, b_ref, h_ref):
    dis = dis_ref[...]
    agg = (p_ref[0] + p_ref[1]) * dis[:, 1:2]
    t = jnp.maximum(agg + b_ref[...], 0.0)
    h = jnp.dot(t, w_ref[...], preferred_element_type=jnp.float32)
    h_ref[...] = h * dis[:, 0:1]


def _mm_fin_body(p_ref, dis_ref, w_ref, b_ref, bfc_ref, o_ref):
    dis = dis_ref[...]
    agg = (p_ref[0] + p_ref[1]) * dis[:, 1:2]
    t = jnp.maximum(agg + b_ref[...], 0.0)
    o_ref[...] = jnp.dot(t, w_ref[...], preferred_element_type=jnp.float32) + bfc_ref[...]


_row_spec = pl.BlockSpec((BR, D), lambda i: (i, 0))
_w_spec = pl.BlockSpec((D, D), lambda i: (0, 0))
_b_spec = pl.BlockSpec((1, D), lambda i: (0, 0))
_dis_spec = pl.BlockSpec((BR, 2), lambda i: (i, 0))
_p_spec = pl.BlockSpec((2, BR, D), lambda i: (0, i, 0))
_deg_spec = pl.BlockSpec((BR, 4), lambda i: (i, 0))

_mm0a = pl.pallas_call(
    _mm0a_body,
    grid=(GRID,),
    in_specs=[_row_spec, _w_spec],
    out_specs=_row_spec,
    out_shape=jax.ShapeDtypeStruct((N, D), jnp.float32),
)

_mm0b = pl.pallas_call(
    _mm0b_body,
    grid=(GRID,),
    in_specs=[_row_spec, _deg_spec],
    out_specs=[_row_spec, _dis_spec],
    out_shape=[
        jax.ShapeDtypeStruct((N, D), jnp.float32),
        jax.ShapeDtypeStruct((N, 2), jnp.float32),
    ],
)

_mm_mid = pl.pallas_call(
    _mm_mid_body,
    grid=(GRID,),
    in_specs=[_p_spec, _dis_spec, _w_spec, _b_spec],
    out_specs=_row_spec,
    out_shape=jax.ShapeDtypeStruct((N, D), jnp.float32),
)

_mm_fin = pl.pallas_call(
    _mm_fin_body,
    grid=(GRID,),
    in_specs=[_p_spec, _dis_spec, _w_spec, _b_spec, _b_spec],
    out_specs=_row_spec,
    out_shape=jax.ShapeDtypeStruct((N, D), jnp.float32),
)


def kernel(features, edge_index, W0, b0, W1, b1, W2, b2, Wfc, bfc):
    e4 = edge_index.reshape(2, NW, NCHUNK, C)
    e2 = edge_index.reshape(2, NW, EPT)
    zn = jnp.zeros((NDP,), jnp.float32)
    zrows = jnp.zeros((N, D), jnp.float32)
    b0r, b1r, b2r, bfcr = (v.reshape(1, D) for v in (b0, b1, b2, bfc))

    deg_parts = _deg_kernel(e4, zn)
    hraw = _mm0a(features, W0)
    deg4 = jnp.transpose(deg_parts.reshape(4, NDP))[:N]
    h, dis = _mm0b(hraw, deg4)
    p = _seg_kernel(h, e2, e4, zrows)
    h = _mm_mid(p, dis, W1, b0r)
    p = _seg_kernel(h, e2, e4, zrows)
    h = _mm_mid(p, dis, W2, b1r)
    p = _seg_kernel(h, e2, e4, zrows)
    return _mm_fin(p, dis, Wfc, b2r, bfcr)


# speedup vs baseline: 1.0782x; 1.0032x over previous
import functools

import jax
import jax.numpy as jnp
from jax import lax
from jax.experimental import pallas as pl
from jax.experimental.pallas import tpu as pltpu
from jax.experimental.pallas import tpu_sc as plsc

N = 10000
NDP = 10240
E = 320000
D = 128

NC = 2
NS = 16
NW = NC * NS
EPT = E // NW
C = 80
NCHUNK = EPT // C
RPT = 624
RPTD = NDP // NS

BR = 2000
GRID = N // BR

_mesh = plsc.VectorSubcoreMesh(core_axis_name="c", subcore_axis_name="s")


@functools.partial(
    pl.kernel,
    out_type=jax.ShapeDtypeStruct((NC, 2, NDP), jnp.float32),
    mesh=_mesh,
    scratch_types=[
        pltpu.VMEM_SHARED((NDP,), jnp.float32),
        pltpu.VMEM_SHARED((NDP,), jnp.float32),
        pltpu.VMEM((NCHUNK, C), jnp.int32),
        pltpu.VMEM((NCHUNK, C), jnp.int32),
        pltpu.VMEM((C,), jnp.float32),
        pltpu.SemaphoreType.DMA,
        pltpu.SemaphoreType.DMA,
        pltpu.SemaphoreType.DMA,
        pltpu.SemaphoreType.DMA,
    ],
)
def _deg_kernel(e4_hbm, zeros_hbm, out_hbm, acc_o, acc_i, sidx, didx, ones,
                d0, d1, d2, d3):
    c = lax.axis_index("c")
    s = lax.axis_index("s")
    tile = c * NS + s
    z0 = pltpu.async_copy(zeros_hbm.at[pl.ds(s * RPTD, RPTD)], acc_o.at[pl.ds(s * RPTD, RPTD)], d0)
    z1 = pltpu.async_copy(zeros_hbm.at[pl.ds(s * RPTD, RPTD)], acc_i.at[pl.ds(s * RPTD, RPTD)], d1)
    i0 = pltpu.async_copy(e4_hbm.at[0, tile], sidx, d2)
    i1 = pltpu.async_copy(e4_hbm.at[1, tile], didx, d3)
    for j in range(C // 16):
        ones[pl.ds(j * 16, 16)] = jnp.full((16,), 1.0, jnp.float32)
    z0.wait(); z1.wait(); i0.wait(); i1.wait()
    plsc.subcore_barrier()

    def _swait(acc, idx, sem):
        pltpu.make_async_copy(ones, acc.at[idx.at[0]], sem).wait()

    pltpu.async_copy(ones, acc_o.at[sidx.at[0]], d0, add=True)
    pltpu.async_copy(ones, acc_i.at[didx.at[0]], d1, add=True)
    pltpu.async_copy(ones, acc_o.at[sidx.at[1]], d2, add=True)
    pltpu.async_copy(ones, acc_i.at[didx.at[1]], d3, add=True)

    def body(j, _):
        k = 2 * j
        _swait(acc_o, sidx, d0)
        pltpu.async_copy(ones, acc_o.at[sidx.at[k]], d0, add=True)
        _swait(acc_i, didx, d1)
        pltpu.async_copy(ones, acc_i.at[didx.at[k]], d1, add=True)
        _swait(acc_o, sidx, d2)
        pltpu.async_copy(ones, acc_o.at[sidx.at[k + 1]], d2, add=True)
        _swait(acc_i, didx, d3)
        pltpu.async_copy(ones, acc_i.at[didx.at[k + 1]], d3, add=True)
        return ()

    lax.fori_loop(1, NCHUNK // 2, body, ())
    la = pltpu.async_copy(ones, acc_o.at[sidx.at[NCHUNK - 1]], d0, add=True)
    lb = pltpu.async_copy(ones, acc_i.at[didx.at[NCHUNK - 1]], d1, add=True)
    _swait(acc_o, sidx, d0); _swait(acc_i, didx, d1)
    _swait(acc_o, sidx, d0); _swait(acc_i, didx, d1)
    _swait(acc_o, sidx, d2); _swait(acc_i, didx, d3)
    plsc.subcore_barrier()
    pltpu.sync_copy(acc_o.at[pl.ds(s * RPTD, RPTD)], out_hbm.at[c, 0, pl.ds(s * RPTD, RPTD)])
    pltpu.sync_copy(acc_i.at[pl.ds(s * RPTD, RPTD)], out_hbm.at[c, 1, pl.ds(s * RPTD, RPTD)])


@functools.partial(
    pl.kernel,
    out_type=jax.ShapeDtypeStruct((NC, N, D), jnp.float32),
    mesh=_mesh,
    scratch_types=[
        pltpu.VMEM_SHARED((N, D), jnp.float32),
        pltpu.VMEM((EPT,), jnp.int32),
        pltpu.VMEM((NCHUNK, C), jnp.int32),
        pltpu.VMEM((C, D), jnp.float32),
        pltpu.VMEM((C, D), jnp.float32),
        pltpu.SemaphoreType.DMA,
        pltpu.SemaphoreType.DMA,
    ],
)
def _seg_kernel(h_hbm, e2_hbm, e4_hbm, zrows_hbm, out_hbm, acc, sidx, didx,
                rows0, rows1, gs0, gs1):
    c = lax.axis_index("c")
    s = lax.axis_index("s")
    tile = c * NS + s
    z0 = pltpu.async_copy(zrows_hbm.at[pl.ds(s * RPT, RPT)], acc.at[pl.ds(s * RPT, RPT)], gs0)
    i0 = pltpu.async_copy(e2_hbm.at[0, tile], sidx, gs1)
    i1 = pltpu.async_copy(e4_hbm.at[1, tile], didx, gs1)

    @pl.when(s == 0)
    def _zero_tail():
        pltpu.async_copy(zrows_hbm.at[pl.ds(NS * RPT, N - NS * RPT)],
                         acc.at[pl.ds(NS * RPT, N - NS * RPT)], gs0).wait()

    z0.wait(); i0.wait(); i1.wait()
    plsc.subcore_barrier()

    def _gwait(buf, sem):
        pltpu.make_async_copy(h_hbm.at[pl.ds(0, C)], buf, sem).wait()

    pltpu.async_copy(h_hbm.at[sidx.at[pl.ds(0, C)]], rows0, gs0)

    def body(j, _):
        k = 2 * j
        pltpu.async_copy(h_hbm.at[sidx.at[pl.ds(pl.multiple_of((k + 1) * C, 8), C)]], rows1, gs1)
        _gwait(rows0, gs0)
        pltpu.sync_copy(rows0, acc.at[didx.at[k]], add=True)
        pltpu.async_copy(h_hbm.at[sidx.at[pl.ds(pl.multiple_of((k + 2) * C, 8), C)]], rows0, gs0)
        _gwait(rows1, gs1)
        pltpu.sync_copy(rows1, acc.at[didx.at[k + 1]], add=True)
        return ()

    lax.fori_loop(0, (NCHUNK - 1) // 2, body, ())
    _gwait(rows0, gs0)
    pltpu.sync_copy(rows0, acc.at[didx.at[NCHUNK - 1]], add=True)
    plsc.subcore_barrier()
    pltpu.sync_copy(acc.at[pl.ds(s * RPT, RPT)], out_hbm.at[c, pl.ds(s * RPT, RPT)])

    @pl.when(s == 0)
    def _out_tail():
        pltpu.sync_copy(acc.at[pl.ds(NS * RPT, N - NS * RPT)],
                        out_hbm.at[c, pl.ds(NS * RPT, N - NS * RPT)])


def _mm0a_body(x_ref, w_ref, h_ref):
    h_ref[...] = jnp.dot(x_ref[...], w_ref[...], preferred_element_type=jnp.float32)


def _mm0b_body(h_ref, deg_ref, ho_ref, dis_ref):
    deg = deg_ref[...]
    deg_out = deg[:, 0:1] + deg[:, 2:3]
    deg_in = deg[:, 1:2] + deg[:, 3:4]
    dis_out = jnp.where(deg_out > 0, lax.rsqrt(jnp.maximum(deg_out, 1.0)), 0.0)
    dis_in = jnp.where(deg_in > 0, lax.rsqrt(jnp.maximum(deg_in, 1.0)), 0.0)
    ho_ref[...] = h_ref[...] * dis_out
    dis_ref[...] = jnp.concatenate([dis_out, dis_in], axis=1)


def _mm_mid_body(p_ref, dis_ref, w_ref, b_ref, h_ref):
    dis = dis_ref[...]
    agg = (p_ref[0] + p_ref[1]) * dis[:, 1:2]
    t = jnp.maximum(agg + b_ref[...], 0.0)
    h = jnp.dot(t, w_ref[...], preferred_element_type=jnp.float32)
    h_ref[...] = h * dis[:, 0:1]


def _mm_fin_body(p_ref, dis_ref, w_ref, b_ref, bfc_ref, o_ref):
    dis = dis_ref[...]
    agg = (p_ref[0] + p_ref[1]) * dis[:, 1:2]
    t = jnp.maximum(agg + b_ref[...], 0.0)
    o_ref[...] = jnp.dot(t, w_ref[...], preferred_element_type=jnp.float32) + bfc_ref[...]


_row_spec = pl.BlockSpec((BR, D), lambda i: (i, 0))
_w_spec = pl.BlockSpec((D, D), lambda i: (0, 0))
_b_spec = pl.BlockSpec((1, D), lambda i: (0, 0))
_dis_spec = pl.BlockSpec((BR, 2), lambda i: (i, 0))
_p_spec = pl.BlockSpec((2, BR, D), lambda i: (0, i, 0))
_deg_spec = pl.BlockSpec((BR, 4), lambda i: (i, 0))

_mm0a = pl.pallas_call(
    _mm0a_body,
    grid=(GRID,),
    in_specs=[_row_spec, _w_spec],
    out_specs=_row_spec,
    out_shape=jax.ShapeDtypeStruct((N, D), jnp.float32),
)

_mm0b = pl.pallas_call(
    _mm0b_body,
    grid=(GRID,),
    in_specs=[_row_spec, _deg_spec],
    out_specs=[_row_spec, _dis_spec],
    out_shape=[
        jax.ShapeDtypeStruct((N, D), jnp.float32),
        jax.ShapeDtypeStruct((N, 2), jnp.float32),
    ],
)

_mm_mid = pl.pallas_call(
    _mm_mid_body,
    grid=(GRID,),
    in_specs=[_p_spec, _dis_spec, _w_spec, _b_spec],
    out_specs=_row_spec,
    out_shape=jax.ShapeDtypeStruct((N, D), jnp.float32),
)

_mm_fin = pl.pallas_call(
    _mm_fin_body,
    grid=(GRID,),
    in_specs=[_p_spec, _dis_spec, _w_spec, _b_spec, _b_spec],
    out_specs=_row_spec,
    out_shape=jax.ShapeDtypeStruct((N, D), jnp.float32),
)


def kernel(features, edge_index, W0, b0, W1, b1, W2, b2, Wfc, bfc):
    e4 = edge_index.reshape(2, NW, NCHUNK, C)
    e2 = edge_index.reshape(2, NW, EPT)
    zn = jnp.zeros((NDP,), jnp.float32)
    zrows = jnp.zeros((N, D), jnp.float32)
    b0r, b1r, b2r, bfcr = (v.reshape(1, D) for v in (b0, b1, b2, bfc))

    deg_parts = _deg_kernel(e4, zn)
    hraw = _mm0a(features, W0)
    deg4 = jnp.transpose(deg_parts.reshape(4, NDP))[:N]
    h, dis = _mm0b(hraw, deg4)
    p = _seg_kernel(h, e2, e4, zrows)
    h = _mm_mid(p, dis, W1, b0r)
    p = _seg_kernel(h, e2, e4, zrows)
    h = _mm_mid(p, dis, W2, b1r)
    p = _seg_kernel(h, e2, e4, zrows)
    return _mm_fin(p, dis, Wfc, b2r, bfcr)


# speedup vs baseline: 1.0879x; 1.0090x over previous
import functools

import jax
import jax.numpy as jnp
from jax import lax
from jax.experimental import pallas as pl
from jax.experimental.pallas import tpu as pltpu
from jax.experimental.pallas import tpu_sc as plsc

N = 10000
NDP = 10240
E = 320000
D = 128

NC = 2
NS = 16
NW = NC * NS
EPT = E // NW
C = 80
NCHUNK = EPT // C
RPT = 624
RPTD = NDP // NS

BR = 2000
GRID = N // BR

_mesh = plsc.VectorSubcoreMesh(core_axis_name="c", subcore_axis_name="s")


@functools.partial(
    pl.kernel,
    out_type=jax.ShapeDtypeStruct((NC, 2, NDP), jnp.float32),
    mesh=_mesh,
    scratch_types=[
        pltpu.VMEM_SHARED((NDP,), jnp.float32),
        pltpu.VMEM_SHARED((NDP,), jnp.float32),
        pltpu.VMEM((NCHUNK, C), jnp.int32),
        pltpu.VMEM((NCHUNK, C), jnp.int32),
        pltpu.VMEM((C,), jnp.float32),
        pltpu.SemaphoreType.DMA,
        pltpu.SemaphoreType.DMA,
        pltpu.SemaphoreType.DMA,
        pltpu.SemaphoreType.DMA,
    ],
)
def _deg_kernel(e4_hbm, zeros_hbm, out_hbm, acc_o, acc_i, sidx, didx, ones,
                d0, d1, d2, d3):
    c = lax.axis_index("c")
    s = lax.axis_index("s")
    tile = c * NS + s
    z0 = pltpu.async_copy(zeros_hbm.at[pl.ds(s * RPTD, RPTD)], acc_o.at[pl.ds(s * RPTD, RPTD)], d0)
    z1 = pltpu.async_copy(zeros_hbm.at[pl.ds(s * RPTD, RPTD)], acc_i.at[pl.ds(s * RPTD, RPTD)], d1)
    i0 = pltpu.async_copy(e4_hbm.at[0, tile], sidx, d2)
    i1 = pltpu.async_copy(e4_hbm.at[1, tile], didx, d3)
    for j in range(C // 16):
        ones[pl.ds(j * 16, 16)] = jnp.full((16,), 1.0, jnp.float32)
    z0.wait(); z1.wait(); i0.wait(); i1.wait()
    plsc.subcore_barrier()

    def _swait(acc, idx, sem):
        pltpu.make_async_copy(ones, acc.at[idx.at[0]], sem).wait()

    pltpu.async_copy(ones, acc_o.at[sidx.at[0]], d0, add=True)
    pltpu.async_copy(ones, acc_i.at[didx.at[0]], d1, add=True)
    pltpu.async_copy(ones, acc_o.at[sidx.at[1]], d2, add=True)
    pltpu.async_copy(ones, acc_i.at[didx.at[1]], d3, add=True)

    def body(j, _):
        k = 2 * j
        _swait(acc_o, sidx, d0)
        pltpu.async_copy(ones, acc_o.at[sidx.at[k]], d0, add=True)
        _swait(acc_i, didx, d1)
        pltpu.async_copy(ones, acc_i.at[didx.at[k]], d1, add=True)
        _swait(acc_o, sidx, d2)
        pltpu.async_copy(ones, acc_o.at[sidx.at[k + 1]], d2, add=True)
        _swait(acc_i, didx, d3)
        pltpu.async_copy(ones, acc_i.at[didx.at[k + 1]], d3, add=True)
        return ()

    lax.fori_loop(1, NCHUNK // 2, body, ())
    la = pltpu.async_copy(ones, acc_o.at[sidx.at[NCHUNK - 1]], d0, add=True)
    lb = pltpu.async_copy(ones, acc_i.at[didx.at[NCHUNK - 1]], d1, add=True)
    _swait(acc_o, sidx, d0); _swait(acc_i, didx, d1)
    _swait(acc_o, sidx, d0); _swait(acc_i, didx, d1)
    _swait(acc_o, sidx, d2); _swait(acc_i, didx, d3)
    plsc.subcore_barrier()
    pltpu.sync_copy(acc_o.at[pl.ds(s * RPTD, RPTD)], out_hbm.at[c, 0, pl.ds(s * RPTD, RPTD)])
    pltpu.sync_copy(acc_i.at[pl.ds(s * RPTD, RPTD)], out_hbm.at[c, 1, pl.ds(s * RPTD, RPTD)])


@functools.partial(
    pl.kernel,
    out_type=jax.ShapeDtypeStruct((NC, N, D), jnp.float32),
    mesh=_mesh,
    scratch_types=[
        pltpu.VMEM_SHARED((N, D), jnp.float32),
        pltpu.VMEM((EPT,), jnp.int32),
        pltpu.VMEM((NCHUNK, C), jnp.int32),
        pltpu.VMEM((C, D), jnp.float32),
        pltpu.VMEM((C, D), jnp.float32),
        pltpu.SemaphoreType.DMA,
        pltpu.SemaphoreType.DMA,
    ],
)
def _seg_kernel(h_hbm, e2_hbm, e4_hbm, zsmall_hbm, out_hbm, acc, sidx, didx,
                rows0, rows1, gs0, gs1):
    c = lax.axis_index("c")
    s = lax.axis_index("s")
    tile = c * NS + s
    zf = pltpu.async_copy(zsmall_hbm, rows0, gs0)
    i0 = pltpu.async_copy(e2_hbm.at[0, tile], sidx, gs1)
    i1 = pltpu.async_copy(e4_hbm.at[1, tile], didx, gs1)
    zf.wait()
    zcs = [pltpu.async_copy(rows0, acc.at[pl.ds(s * RPT + r * C, C)], gs0)
           for r in range(RPT // C)]
    zcs.append(pltpu.async_copy(rows0.at[pl.ds(0, RPT - C * (RPT // C))],
                                acc.at[pl.ds(s * RPT + C * (RPT // C), RPT - C * (RPT // C))],
                                gs0))

    @pl.when(s == 0)
    def _zero_tail():
        pltpu.async_copy(rows0.at[pl.ds(0, N - NS * RPT)],
                         acc.at[pl.ds(NS * RPT, N - NS * RPT)], gs0).wait()

    for z in zcs:
        z.wait()
    i0.wait(); i1.wait()
    plsc.subcore_barrier()

    def _gwait(buf, sem):
        pltpu.make_async_copy(h_hbm.at[pl.ds(0, C)], buf, sem).wait()

    pltpu.async_copy(h_hbm.at[sidx.at[pl.ds(0, C)]], rows0, gs0)

    def body(j, _):
        k = 2 * j
        pltpu.async_copy(h_hbm.at[sidx.at[pl.ds(pl.multiple_of((k + 1) * C, 8), C)]], rows1, gs1)
        _gwait(rows0, gs0)
        pltpu.sync_copy(rows0, acc.at[didx.at[k]], add=True)
        pltpu.async_copy(h_hbm.at[sidx.at[pl.ds(pl.multiple_of((k + 2) * C, 8), C)]], rows0, gs0)
        _gwait(rows1, gs1)
        pltpu.sync_copy(rows1, acc.at[didx.at[k + 1]], add=True)
        return ()

    lax.fori_loop(0, (NCHUNK - 1) // 2, body, ())
    _gwait(rows0, gs0)
    pltpu.sync_copy(rows0, acc.at[didx.at[NCHUNK - 1]], add=True)
    plsc.subcore_barrier()
    pltpu.sync_copy(acc.at[pl.ds(s * RPT, RPT)], out_hbm.at[c, pl.ds(s * RPT, RPT)])

    @pl.when(s == 0)
    def _out_tail():
        pltpu.sync_copy(acc.at[pl.ds(NS * RPT, N - NS * RPT)],
                        out_hbm.at[c, pl.ds(NS * RPT, N - NS * RPT)])


def _mm0a_body(x_ref, w_ref, h_ref):
    h_ref[...] = jnp.dot(x_ref[...], w_ref[...], preferred_element_type=jnp.float32)


def _mm0b_body(h_ref, deg_ref, ho_ref, dis_ref):
    deg = deg_ref[...]
    deg_out = deg[:, 0:1] + deg[:, 2:3]
    deg_in = deg[:, 1:2] + deg[:, 3:4]
    dis_out = jnp.where(deg_out > 0, lax.rsqrt(jnp.maximum(deg_out, 1.0)), 0.0)
    dis_in = jnp.where(deg_in > 0, lax.rsqrt(jnp.maximum(deg_in, 1.0)), 0.0)
    ho_ref[...] = h_ref[...] * dis_out
    dis_ref[...] = jnp.concatenate([dis_out, dis_in], axis=1)


def _mm_mid_body(p_ref, dis_ref, w_ref, b_ref, h_ref):
    dis = dis_ref[...]
    agg = (p_ref[0] + p_ref[1]) * dis[:, 1:2]
    t = jnp.maximum(agg + b_ref[...], 0.0)
    h = jnp.dot(t, w_ref[...], preferred_element_type=jnp.float32)
    h_ref[...] = h * dis[:, 0:1]


def _mm_fin_body(p_ref, dis_ref, w_ref, b_ref, bfc_ref, o_ref):
    dis = dis_ref[...]
    agg = (p_ref[0] + p_ref[1]) * dis[:, 1:2]
    t = jnp.maximum(agg + b_ref[...], 0.0)
    o_ref[...] = jnp.dot(t, w_ref[...], preferred_element_type=jnp.float32) + bfc_ref[...]


_row_spec = pl.BlockSpec((BR, D), lambda i: (i, 0))
_w_spec = pl.BlockSpec((D, D), lambda i: (0, 0))
_b_spec = pl.BlockSpec((1, D), lambda i: (0, 0))
_dis_spec = pl.BlockSpec((BR, 2), lambda i: (i, 0))
_p_spec = pl.BlockSpec((2, BR, D), lambda i: (0, i, 0))
_deg_spec = pl.BlockSpec((BR, 4), lambda i: (i, 0))

_mm0a = pl.pallas_call(
    _mm0a_body,
    grid=(GRID,),
    in_specs=[_row_spec, _w_spec],
    out_specs=_row_spec,
    out_shape=jax.ShapeDtypeStruct((N, D), jnp.float32),
)

_mm0b = pl.pallas_call(
    _mm0b_body,
    grid=(GRID,),
    in_specs=[_row_spec, _deg_spec],
    out_specs=[_row_spec, _dis_spec],
    out_shape=[
        jax.ShapeDtypeStruct((N, D), jnp.float32),
        jax.ShapeDtypeStruct((N, 2), jnp.float32),
    ],
)

_mm_mid = pl.pallas_call(
    _mm_mid_body,
    grid=(GRID,),
    in_specs=[_p_spec, _dis_spec, _w_spec, _b_spec],
    out_specs=_row_spec,
    out_shape=jax.ShapeDtypeStruct((N, D), jnp.float32),
)

_mm_fin = pl.pallas_call(
    _mm_fin_body,
    grid=(GRID,),
    in_specs=[_p_spec, _dis_spec, _w_spec, _b_spec, _b_spec],
    out_specs=_row_spec,
    out_shape=jax.ShapeDtypeStruct((N, D), jnp.float32),
)


def kernel(features, edge_index, W0, b0, W1, b1, W2, b2, Wfc, bfc):
    e4 = edge_index.reshape(2, NW, NCHUNK, C)
    e2 = edge_index.reshape(2, NW, EPT)
    zn = jnp.zeros((NDP,), jnp.float32)
    zsmall = jnp.zeros((C, D), jnp.float32)
    b0r, b1r, b2r, bfcr = (v.reshape(1, D) for v in (b0, b1, b2, bfc))

    deg_parts = _deg_kernel(e4, zn)
    hraw = _mm0a(features, W0)
    deg4 = jnp.transpose(deg_parts.reshape(4, NDP))[:N]
    h, dis = _mm0b(hraw, deg4)
    p = _seg_kernel(h, e2, e4, zsmall)
    h = _mm_mid(p, dis, W1, b0r)
    p = _seg_kernel(h, e2, e4, zsmall)
    h = _mm_mid(p, dis, W2, b1r)
    p = _seg_kernel(h, e2, e4, zsmall)
    return _mm_fin(p, dis, Wfc, b2r, bfcr)


# speedup vs baseline: 1.1103x; 1.0206x over previous
import functools

import jax
import jax.numpy as jnp
from jax import lax
from jax.experimental import pallas as pl
from jax.experimental.pallas import tpu as pltpu
from jax.experimental.pallas import tpu_sc as plsc

N = 10000
NDP = 10240
E = 320000
D = 128

NC = 2
NS = 16
NW = NC * NS
EPT = E // NW
C = 80
NCHUNK = EPT // C
RPT = 624
RPTD = NDP // NS

BR = 5000
GRID = N // BR

_mesh = plsc.VectorSubcoreMesh(core_axis_name="c", subcore_axis_name="s")


@functools.partial(
    pl.kernel,
    out_type=jax.ShapeDtypeStruct((NC, 2, NDP), jnp.float32),
    mesh=_mesh,
    scratch_types=[
        pltpu.VMEM_SHARED((NDP,), jnp.float32),
        pltpu.VMEM_SHARED((NDP,), jnp.float32),
        pltpu.VMEM((NCHUNK, C), jnp.int32),
        pltpu.VMEM((NCHUNK, C), jnp.int32),
        pltpu.VMEM((C,), jnp.float32),
        pltpu.SemaphoreType.DMA,
        pltpu.SemaphoreType.DMA,
        pltpu.SemaphoreType.DMA,
        pltpu.SemaphoreType.DMA,
    ],
)
def _deg_kernel(e4_hbm, zeros_hbm, out_hbm, acc_o, acc_i, sidx, didx, ones,
                d0, d1, d2, d3):
    c = lax.axis_index("c")
    s = lax.axis_index("s")
    tile = c * NS + s
    z0 = pltpu.async_copy(zeros_hbm.at[pl.ds(s * RPTD, RPTD)], acc_o.at[pl.ds(s * RPTD, RPTD)], d0)
    z1 = pltpu.async_copy(zeros_hbm.at[pl.ds(s * RPTD, RPTD)], acc_i.at[pl.ds(s * RPTD, RPTD)], d1)
    i0 = pltpu.async_copy(e4_hbm.at[0, tile], sidx, d2)
    i1 = pltpu.async_copy(e4_hbm.at[1, tile], didx, d3)
    for j in range(C // 16):
        ones[pl.ds(j * 16, 16)] = jnp.full((16,), 1.0, jnp.float32)
    z0.wait(); z1.wait(); i0.wait(); i1.wait()
    plsc.subcore_barrier()

    def _swait(acc, idx, sem):
        pltpu.make_async_copy(ones, acc.at[idx.at[0]], sem).wait()

    pltpu.async_copy(ones, acc_o.at[sidx.at[0]], d0, add=True)
    pltpu.async_copy(ones, acc_i.at[didx.at[0]], d1, add=True)
    pltpu.async_copy(ones, acc_o.at[sidx.at[1]], d2, add=True)
    pltpu.async_copy(ones, acc_i.at[didx.at[1]], d3, add=True)

    def body(j, _):
        k = 2 * j
        _swait(acc_o, sidx, d0)
        pltpu.async_copy(ones, acc_o.at[sidx.at[k]], d0, add=True)
        _swait(acc_i, didx, d1)
        pltpu.async_copy(ones, acc_i.at[didx.at[k]], d1, add=True)
        _swait(acc_o, sidx, d2)
        pltpu.async_copy(ones, acc_o.at[sidx.at[k + 1]], d2, add=True)
        _swait(acc_i, didx, d3)
        pltpu.async_copy(ones, acc_i.at[didx.at[k + 1]], d3, add=True)
        return ()

    lax.fori_loop(1, NCHUNK // 2, body, ())
    la = pltpu.async_copy(ones, acc_o.at[sidx.at[NCHUNK - 1]], d0, add=True)
    lb = pltpu.async_copy(ones, acc_i.at[didx.at[NCHUNK - 1]], d1, add=True)
    _swait(acc_o, sidx, d0); _swait(acc_i, didx, d1)
    _swait(acc_o, sidx, d0); _swait(acc_i, didx, d1)
    _swait(acc_o, sidx, d2); _swait(acc_i, didx, d3)
    plsc.subcore_barrier()
    pltpu.sync_copy(acc_o.at[pl.ds(s * RPTD, RPTD)], out_hbm.at[c, 0, pl.ds(s * RPTD, RPTD)])
    pltpu.sync_copy(acc_i.at[pl.ds(s * RPTD, RPTD)], out_hbm.at[c, 1, pl.ds(s * RPTD, RPTD)])


@functools.partial(
    pl.kernel,
    out_type=jax.ShapeDtypeStruct((NC, N, D), jnp.float32),
    mesh=_mesh,
    scratch_types=[
        pltpu.VMEM_SHARED((N, D), jnp.float32),
        pltpu.VMEM((EPT,), jnp.int32),
        pltpu.VMEM((NCHUNK, C), jnp.int32),
        pltpu.VMEM((C, D), jnp.float32),
        pltpu.VMEM((C, D), jnp.float32),
        pltpu.SemaphoreType.DMA,
        pltpu.SemaphoreType.DMA,
    ],
)
def _seg_kernel(h_hbm, e2_hbm, e4_hbm, zsmall_hbm, out_hbm, acc, sidx, didx,
                rows0, rows1, gs0, gs1):
    c = lax.axis_index("c")
    s = lax.axis_index("s")
    tile = c * NS + s
    zf = pltpu.async_copy(zsmall_hbm, rows0, gs0)
    i0 = pltpu.async_copy(e2_hbm.at[0, tile], sidx, gs1)
    i1 = pltpu.async_copy(e4_hbm.at[1, tile], didx, gs1)
    zf.wait()
    zcs = [pltpu.async_copy(rows0, acc.at[pl.ds(s * RPT + r * C, C)], gs0)
           for r in range(RPT // C)]
    zcs.append(pltpu.async_copy(rows0.at[pl.ds(0, RPT - C * (RPT // C))],
                                acc.at[pl.ds(s * RPT + C * (RPT // C), RPT - C * (RPT // C))],
                                gs0))

    @pl.when(s == 0)
    def _zero_tail():
        pltpu.async_copy(rows0.at[pl.ds(0, N - NS * RPT)],
                         acc.at[pl.ds(NS * RPT, N - NS * RPT)], gs0).wait()

    for z in zcs:
        z.wait()
    i0.wait(); i1.wait()
    plsc.subcore_barrier()

    def _gwait(buf, sem):
        pltpu.make_async_copy(h_hbm.at[pl.ds(0, C)], buf, sem).wait()

    pltpu.async_copy(h_hbm.at[sidx.at[pl.ds(0, C)]], rows0, gs0)

    def body(j, _):
        k = 2 * j
        pltpu.async_copy(h_hbm.at[sidx.at[pl.ds(pl.multiple_of((k + 1) * C, 8), C)]], rows1, gs1)
        _gwait(rows0, gs0)
        pltpu.sync_copy(rows0, acc.at[didx.at[k]], add=True)
        pltpu.async_copy(h_hbm.at[sidx.at[pl.ds(pl.multiple_of((k + 2) * C, 8), C)]], rows0, gs0)
        _gwait(rows1, gs1)
        pltpu.sync_copy(rows1, acc.at[didx.at[k + 1]], add=True)
        return ()

    lax.fori_loop(0, (NCHUNK - 1) // 2, body, ())
    _gwait(rows0, gs0)
    pltpu.sync_copy(rows0, acc.at[didx.at[NCHUNK - 1]], add=True)
    plsc.subcore_barrier()
    pltpu.sync_copy(acc.at[pl.ds(s * RPT, RPT)], out_hbm.at[c, pl.ds(s * RPT, RPT)])

    @pl.when(s == 0)
    def _out_tail():
        pltpu.sync_copy(acc.at[pl.ds(NS * RPT, N - NS * RPT)],
                        out_hbm.at[c, pl.ds(NS * RPT, N - NS * RPT)])


def _mm0a_body(x_ref, w_ref, h_ref):
    h_ref[...] = jnp.dot(x_ref[...], w_ref[...], preferred_element_type=jnp.float32)


def _mm0b_body(h_ref, deg_ref, ho_ref, dis_ref):
    deg = deg_ref[...]
    deg_out = deg[:, 0:1] + deg[:, 2:3]
    deg_in = deg[:, 1:2] + deg[:, 3:4]
    dis_out = jnp.where(deg_out > 0, lax.rsqrt(jnp.maximum(deg_out, 1.0)), 0.0)
    dis_in = jnp.where(deg_in > 0, lax.rsqrt(jnp.maximum(deg_in, 1.0)), 0.0)
    ho_ref[...] = h_ref[...] * dis_out
    dis_ref[...] = jnp.concatenate([dis_out, dis_in], axis=1)


def _mm_mid_body(p_ref, dis_ref, w_ref, b_ref, h_ref):
    dis = dis_ref[...]
    agg = (p_ref[0] + p_ref[1]) * dis[:, 1:2]
    t = jnp.maximum(agg + b_ref[...], 0.0)
    h = jnp.dot(t, w_ref[...], preferred_element_type=jnp.float32)
    h_ref[...] = h * dis[:, 0:1]


def _mm_fin_body(p_ref, dis_ref, w_ref, b_ref, bfc_ref, o_ref):
    dis = dis_ref[...]
    agg = (p_ref[0] + p_ref[1]) * dis[:, 1:2]
    t = jnp.maximum(agg + b_ref[...], 0.0)
    o_ref[...] = jnp.dot(t, w_ref[...], preferred_element_type=jnp.float32) + bfc_ref[...]


_row_spec = pl.BlockSpec((BR, D), lambda i: (i, 0))
_w_spec = pl.BlockSpec((D, D), lambda i: (0, 0))
_b_spec = pl.BlockSpec((1, D), lambda i: (0, 0))
_dis_spec = pl.BlockSpec((BR, 2), lambda i: (i, 0))
_p_spec = pl.BlockSpec((2, BR, D), lambda i: (0, i, 0))
_deg_spec = pl.BlockSpec((BR, 4), lambda i: (i, 0))

_mm0a = pl.pallas_call(
    _mm0a_body,
    grid=(GRID,),
    in_specs=[_row_spec, _w_spec],
    out_specs=_row_spec,
    out_shape=jax.ShapeDtypeStruct((N, D), jnp.float32),
)

_mm0b = pl.pallas_call(
    _mm0b_body,
    grid=(GRID,),
    in_specs=[_row_spec, _deg_spec],
    out_specs=[_row_spec, _dis_spec],
    out_shape=[
        jax.ShapeDtypeStruct((N, D), jnp.float32),
        jax.ShapeDtypeStruct((N, 2), jnp.float32),
    ],
)

_mm_mid = pl.pallas_call(
    _mm_mid_body,
    grid=(GRID,),
    in_specs=[_p_spec, _dis_spec, _w_spec, _b_spec],
    out_specs=_row_spec,
    out_shape=jax.ShapeDtypeStruct((N, D), jnp.float32),
)

_mm_fin = pl.pallas_call(
    _mm_fin_body,
    grid=(GRID,),
    in_specs=[_p_spec, _dis_spec, _w_spec, _b_spec, _b_spec],
    out_specs=_row_spec,
    out_shape=jax.ShapeDtypeStruct((N, D), jnp.float32),
)


def kernel(features, edge_index, W0, b0, W1, b1, W2, b2, Wfc, bfc):
    e4 = edge_index.reshape(2, NW, NCHUNK, C)
    e2 = edge_index.reshape(2, NW, EPT)
    zn = jnp.zeros((NDP,), jnp.float32)
    zsmall = jnp.zeros((C, D), jnp.float32)
    b0r, b1r, b2r, bfcr = (v.reshape(1, D) for v in (b0, b1, b2, bfc))

    deg_parts = _deg_kernel(e4, zn)
    hraw = _mm0a(features, W0)
    deg4 = jnp.transpose(deg_parts.reshape(4, NDP))[:N]
    h, dis = _mm0b(hraw, deg4)
    p = _seg_kernel(h, e2, e4, zsmall)
    h = _mm_mid(p, dis, W1, b0r)
    p = _seg_kernel(h, e2, e4, zsmall)
    h = _mm_mid(p, dis, W2, b1r)
    p = _seg_kernel(h, e2, e4, zsmall)
    return _mm_fin(p, dis, Wfc, b2r, bfcr)
